# Initial kernel scaffold; baseline (speedup 1.0000x reference)
import jax
import jax.numpy as jnp
from jax.experimental import pallas as pl


def kernel(mlvl_feats, reference_points, lidar2img, W1, b1, W2, b2):
    raise NotImplementedError("write your pallas kernel here")


# speedup vs baseline: 22.9277x; 22.9277x over previous
import functools

import jax
import jax.numpy as jnp
from jax.experimental import pallas as pl

PC_RANGE = [-51.2, -51.2, -5.0, 51.2, 51.2, 3.0]
IMG_H, IMG_W = 256, 704
EPS = 1e-06

N_CAM = 6
HF, WF = 32, 88
HW = HF * WF
C = 256
R_TOT = 64 * 64 * 4
TILE_R = 256


def _bev_kernel(rp_ref, pts4_ref, l2it_ref, fm_ref, w1_ref, b1_ref, w2_ref,
                b2_ref, out_ref, uv_ref):
    rp = rp_ref[...]
    mid = jnp.maximum(
        jnp.dot(rp, w1_ref[...], preferred_element_type=jnp.float32)
        + b1_ref[...][None, :], 0.0)
    acc = (jnp.dot(mid, w2_ref[...], preferred_element_type=jnp.float32)
           + b2_ref[...][None, :])

    pts4 = pts4_ref[...]

    coli = jax.lax.broadcasted_iota(jnp.int32, (1, HW), 1)
    hci = coli // WF
    hc = hci.astype(jnp.float32)
    xc = (coli - hci * WF).astype(jnp.float32)

    for n in range(N_CAM):
        pc = jnp.dot(pts4, l2it_ref[n], preferred_element_type=jnp.float32)
        pc0 = pc[:, 0:1]
        pc1 = pc[:, 1:2]
        depth = pc[:, 2:3]
        maxd = jnp.maximum(depth, EPS)
        u = (pc0 / maxd) / IMG_W
        v = (pc1 / maxd) / IMG_H
        uv_ref[:, 2 * n:2 * n + 1] = u
        uv_ref[:, 2 * n + 1:2 * n + 2] = v
        mask = ((depth > EPS) & (u >= 0.0) & (u <= 1.0)
                & (v >= 0.0) & (v <= 1.0))
        x = u * WF - 0.5
        y = v * HF - 0.5
        y = jnp.where(mask, y, -1e9)
        a = (jnp.maximum(1.0 - jnp.abs(y - hc), 0.0)
             * jnp.maximum(1.0 - jnp.abs(x - xc), 0.0))
        acc = acc + jnp.dot(a, fm_ref[n], preferred_element_type=jnp.float32)

    out_ref[...] = acc


@jax.jit
def kernel(mlvl_feats, reference_points, lidar2img, W1, b1, W2, b2):
    B = mlvl_feats.shape[0]
    rp_flat = reference_points[0].transpose(1, 2, 0, 3).reshape(R_TOT, 3)
    pts4 = jnp.concatenate([
        rp_flat[:, 0:1] * (PC_RANGE[3] - PC_RANGE[0]) + PC_RANGE[0],
        rp_flat[:, 1:2] * (PC_RANGE[4] - PC_RANGE[1]) + PC_RANGE[1],
        rp_flat[:, 2:3] * (PC_RANGE[5] - PC_RANGE[2]) + PC_RANGE[2],
        jnp.ones((R_TOT, 1), jnp.float32),
    ], axis=-1)
    fm = mlvl_feats[0].transpose(0, 2, 3, 1).reshape(N_CAM, HW, C)
    l2it = jnp.transpose(lidar2img[0], (0, 2, 1))

    grid = (R_TOT // TILE_R,)
    out, uv = pl.pallas_call(
        _bev_kernel,
        grid=grid,
        in_specs=[
            pl.BlockSpec((TILE_R, 3), lambda i: (i, 0)),
            pl.BlockSpec((TILE_R, 4), lambda i: (i, 0)),
            pl.BlockSpec((N_CAM, 4, 4), lambda i: (0, 0, 0)),
            pl.BlockSpec((N_CAM, HW, C), lambda i: (0, 0, 0)),
            pl.BlockSpec((3, 512), lambda i: (0, 0)),
            pl.BlockSpec((512,), lambda i: (0,)),
            pl.BlockSpec((512, C), lambda i: (0, 0)),
            pl.BlockSpec((C,), lambda i: (0,)),
        ],
        out_specs=[
            pl.BlockSpec((TILE_R, C), lambda i: (i, 0)),
            pl.BlockSpec((TILE_R, 2 * N_CAM), lambda i: (i, 0)),
        ],
        out_shape=[
            jax.ShapeDtypeStruct((R_TOT, C), jnp.float32),
            jax.ShapeDtypeStruct((R_TOT, 2 * N_CAM), jnp.float32),
        ],
    )(rp_flat, pts4, l2it, fm, W1, b1, W2, b2)

    sf = out.reshape(1, 64, 64, 4, C).transpose(0, 4, 3, 1, 2)
    spc = uv.reshape(4096, 4, N_CAM, 2).transpose(2, 0, 1, 3)
    spc = spc.reshape(1, N_CAM, 4096, 1, 4, 2)
    return sf, spc


# speedup vs baseline: 23.3116x; 1.0167x over previous
import functools

import jax
import jax.numpy as jnp
from jax.experimental import pallas as pl

PC_RANGE = [-51.2, -51.2, -5.0, 51.2, 51.2, 3.0]
IMG_H, IMG_W = 256, 704
EPS = 1e-06

N_CAM = 6
HF, WF = 32, 88
HW = HF * WF
C = 256
R_TOT = 64 * 64 * 4
TILE_R = 256


def _bev_kernel(rp_ref, pts4_ref, l2it_ref, fm_ref, w1_ref, b1_ref, w2_ref,
                b2_ref, out_ref, uv_ref):
    rp = rp_ref[...]
    mid = jnp.maximum(
        jnp.dot(rp, w1_ref[...], preferred_element_type=jnp.float32)
        + b1_ref[...][None, :], 0.0)
    acc = (jnp.dot(mid, w2_ref[...], preferred_element_type=jnp.float32)
           + b2_ref[...][None, :])

    pts4 = pts4_ref[...]

    coli = jax.lax.broadcasted_iota(jnp.int32, (1, HW), 1)
    hci = coli // WF
    hc = hci.astype(jnp.float32)
    xc = (coli - hci * WF).astype(jnp.float32)

    for n in range(N_CAM):
        pc = jnp.dot(pts4, l2it_ref[n], preferred_element_type=jnp.float32)
        pc0 = pc[:, 0:1]
        pc1 = pc[:, 1:2]
        depth = pc[:, 2:3]
        maxd = jnp.maximum(depth, EPS)
        u = (pc0 / maxd) / IMG_W
        v = (pc1 / maxd) / IMG_H
        uv_ref[:, 2 * n:2 * n + 1] = u
        uv_ref[:, 2 * n + 1:2 * n + 2] = v
        mask = ((depth > EPS) & (u >= 0.0) & (u <= 1.0)
                & (v >= 0.0) & (v <= 1.0))
        x = u * WF - 0.5
        y = v * HF - 0.5
        y = jnp.where(mask, y, -1e9)
        a = (jnp.maximum(1.0 - jnp.abs(y - hc), 0.0)
             * jnp.maximum(1.0 - jnp.abs(x - xc), 0.0))
        acc = acc + jnp.dot(a.astype(jnp.bfloat16), fm_ref[n],
                            preferred_element_type=jnp.float32)

    out_ref[...] = acc


@jax.jit
def kernel(mlvl_feats, reference_points, lidar2img, W1, b1, W2, b2):
    B = mlvl_feats.shape[0]
    rp_flat = reference_points[0].transpose(1, 2, 0, 3).reshape(R_TOT, 3)
    pts4 = jnp.concatenate([
        rp_flat[:, 0:1] * (PC_RANGE[3] - PC_RANGE[0]) + PC_RANGE[0],
        rp_flat[:, 1:2] * (PC_RANGE[4] - PC_RANGE[1]) + PC_RANGE[1],
        rp_flat[:, 2:3] * (PC_RANGE[5] - PC_RANGE[2]) + PC_RANGE[2],
        jnp.ones((R_TOT, 1), jnp.float32),
    ], axis=-1)
    fm = mlvl_feats[0].transpose(0, 2, 3, 1).reshape(N_CAM, HW, C)
    fm = fm.astype(jnp.bfloat16)
    l2it = jnp.transpose(lidar2img[0], (0, 2, 1))

    grid = (R_TOT // TILE_R,)
    out, uv = pl.pallas_call(
        _bev_kernel,
        grid=grid,
        in_specs=[
            pl.BlockSpec((TILE_R, 3), lambda i: (i, 0)),
            pl.BlockSpec((TILE_R, 4), lambda i: (i, 0)),
            pl.BlockSpec((N_CAM, 4, 4), lambda i: (0, 0, 0)),
            pl.BlockSpec((N_CAM, HW, C), lambda i: (0, 0, 0)),
            pl.BlockSpec((3, 512), lambda i: (0, 0)),
            pl.BlockSpec((512,), lambda i: (0,)),
            pl.BlockSpec((512, C), lambda i: (0, 0)),
            pl.BlockSpec((C,), lambda i: (0,)),
        ],
        out_specs=[
            pl.BlockSpec((TILE_R, C), lambda i: (i, 0)),
            pl.BlockSpec((TILE_R, 2 * N_CAM), lambda i: (i, 0)),
        ],
        out_shape=[
            jax.ShapeDtypeStruct((R_TOT, C), jnp.float32),
            jax.ShapeDtypeStruct((R_TOT, 2 * N_CAM), jnp.float32),
        ],
    )(rp_flat, pts4, l2it, fm, W1, b1, W2, b2)

    sf = out.reshape(1, 64, 64, 4, C).transpose(0, 4, 3, 1, 2)
    spc = uv.reshape(4096, 4, N_CAM, 2).transpose(2, 0, 1, 3)
    spc = spc.reshape(1, N_CAM, 4096, 1, 4, 2)
    return sf, spc
